# baseline (speedup 1.0000x reference)
import functools

import jax
import jax.numpy as jnp
from jax import lax
from jax.experimental import pallas as pl
from jax.experimental.pallas import tpu as pltpu
from jax.experimental.pallas import tpu_sc as plsc

D = 64
T = 200
B = 4096
W = 128
NW = 32
SCALE = 8.0
GP = 144


def _sc_gather(lut2, xt):
    mesh = plsc.VectorSubcoreMesh(core_axis_name="c", subcore_axis_name="s")

    @functools.partial(
        pl.kernel,
        out_type=jax.ShapeDtypeStruct((T, D, B), jnp.float32),
        mesh=mesh,
        scratch_types=[
            pltpu.VMEM((T, W), jnp.int32),
            pltpu.VMEM((2, W), jnp.int32),
            pltpu.VMEM((2, W), jnp.int32),
            pltpu.VMEM((2, W, GP), jnp.float32),
            pltpu.VMEM((2, D, W), jnp.float32),
            pltpu.SemaphoreType.DMA,
            pltpu.SemaphoreType.DMA,
            pltpu.SemaphoreType.DMA,
            pltpu.SemaphoreType.DMA,
            pltpu.SemaphoreType.DMA,
        ],
        compiler_params=pltpu.CompilerParams(needs_layout_passes=False),
    )
    def k(lut_hbm, xt_hbm, out_hbm, idx_all, rid, colb, g, tr, gs0, gs1,
          ws0, ws1, isem):
        w = lax.axis_index("s") * 2 + lax.axis_index("c")
        b0 = w * W
        gsem = (gs0, gs1)
        wsem = (ws0, ws1)
        riota = lax.iota(jnp.int32, 16)

        pltpu.async_copy(xt_hbm.at[:, pl.ds(b0, W)], idx_all, isem).wait()

        def prep(step, q):
            for j in range(0, W, 16):
                iv = idx_all[step, pl.ds(j, 16)]
                rid[q, pl.ds(j, 16)] = jnp.right_shift(iv, 1)
                colb[q, pl.ds(j, 16)] = jnp.left_shift(
                    jnp.bitwise_and(iv, 1), 6)

        def start_gather(q):
            pltpu.make_async_copy(
                lut_hbm.at[rid.at[q]], g.at[q, :, pl.ds(0, 2 * D)],
                gsem[q]).start()

        def wait_gather(q):
            pltpu.make_async_copy(
                lut_hbm.at[rid.at[q]], g.at[q, :, pl.ds(0, 2 * D)],
                gsem[q]).wait()

        def start_write(step, q):
            pltpu.make_async_copy(
                tr.at[q], out_hbm.at[step, :, pl.ds(b0, W)], wsem[q]).start()

        def wait_write(step, q):
            pltpu.make_async_copy(
                tr.at[q], out_hbm.at[step, :, pl.ds(b0, W)], wsem[q]).wait()

        def quads(q):
            @plsc.parallel_loop(0, W, step=16)
            def _(jg):
                row16 = riota + jg
                colb16 = colb[q, pl.ds(jg, 16)]

                @plsc.parallel_loop(0, D, step=1, unroll=16)
                def _(d):
                    col = colb16 + d
                    v = plsc.load_gather(g.at[q], [row16, col])
                    tr.at[q, d, pl.ds(jg, 16)][...] = v * SCALE

        prep(0, 0)
        start_gather(0)

        @pl.loop(0, T, step=2)
        def _(k0):
            for p in (0, 1):
                kk = k0 + p
                q = 1 - p

                @pl.when(kk < T - 1)
                def _():
                    prep(kk + 1, q)
                    start_gather(q)

                wait_gather(p)

                @pl.when(kk >= 2)
                def _():
                    wait_write(kk - 2, p)

                quads(p)
                start_write(kk, p)

        wait_write(T - 2, 0)
        wait_write(T - 1, 1)

    return k(lut2, xt)


def kernel(x, lut):
    xt = x.T
    lut2 = lut.reshape(lut.shape[0] // 2, 2 * D)
    out_t = _sc_gather(lut2, xt)
    return out_t.transpose(2, 0, 1)


# speedup vs baseline: 1.0960x; 1.0960x over previous
import functools

import jax
import jax.numpy as jnp
from jax import lax
from jax.experimental import pallas as pl
from jax.experimental.pallas import tpu as pltpu
from jax.experimental.pallas import tpu_sc as plsc

D = 64
N = 819200
W = 128
NW = 32
PW = N // NW
TS = PW // W
SCALE = 8.0


def _sc_gather(lut, xin):
    mesh = plsc.VectorSubcoreMesh(core_axis_name="c", subcore_axis_name="s")

    @functools.partial(
        pl.kernel,
        out_type=jax.ShapeDtypeStruct((N, D), jnp.float32),
        mesh=mesh,
        scratch_types=[
            pltpu.VMEM((PW,), jnp.int32),
            pltpu.VMEM((2, W, D), jnp.float32),
            pltpu.VMEM((2, W, D), jnp.float32),
            pltpu.SemaphoreType.DMA,
            pltpu.SemaphoreType.DMA,
            pltpu.SemaphoreType.DMA,
            pltpu.SemaphoreType.DMA,
            pltpu.SemaphoreType.DMA,
        ],
        compiler_params=pltpu.CompilerParams(use_tc_tiling_on_sc=False),
    )
    def k(lut_hbm, x_hbm, out_hbm, idx_all, g, tr, gs0, gs1, ws0, ws1, isem):
        w = lax.axis_index("s") * 2 + lax.axis_index("c")
        base = w * PW
        gsem = (gs0, gs1)
        wsem = (ws0, ws1)

        pltpu.async_copy(x_hbm.at[0, pl.ds(base, PW)], idx_all, isem).wait()

        def start_gather(step, q):
            pltpu.make_async_copy(
                lut_hbm.at[idx_all.at[pl.ds(step * W, W)]], g.at[q],
                gsem[q]).start()

        def wait_gather(step, q):
            pltpu.make_async_copy(
                lut_hbm.at[idx_all.at[pl.ds(step * W, W)]], g.at[q],
                gsem[q]).wait()

        def start_write(step, q):
            pltpu.make_async_copy(
                tr.at[q], out_hbm.at[pl.ds(base + step * W, W), :],
                wsem[q]).start()

        def wait_write(step, q):
            pltpu.make_async_copy(
                tr.at[q], out_hbm.at[pl.ds(base + step * W, W), :],
                wsem[q]).wait()

        def scale(q):
            @plsc.parallel_loop(0, W, step=1, unroll=8)
            def _(j):
                for c in range(0, D, 16):
                    v = g[q, j, pl.ds(c, 16)]
                    tr.at[q, j, pl.ds(c, 16)][...] = v * SCALE

        start_gather(0, 0)

        @pl.loop(0, TS, step=2)
        def _(k0):
            for p in (0, 1):
                kk = k0 + p
                q = 1 - p

                @pl.when(kk < TS - 1)
                def _():
                    start_gather(kk + 1, q)

                wait_gather(kk, p)

                @pl.when(kk >= 2)
                def _():
                    wait_write(kk - 2, p)

                scale(p)
                start_write(kk, p)

        wait_write(TS - 2, 0)
        wait_write(TS - 1, 1)

    return k(lut, xin)


def kernel(x, lut):
    xin = x.reshape(1, N)
    out = _sc_gather(lut, xin)
    return out.reshape(x.shape[0], x.shape[1], D)
